# baseline (speedup 1.0000x reference)
import functools

import jax
import jax.numpy as jnp
from jax import lax
from jax.experimental import pallas as pl
from jax.experimental.pallas import tpu as pltpu
from jax.experimental.pallas import tpu_sc as plsc

VOCAB = 1000000
EMBED_DIM = 64
BATCH = 16384
HIST = 50

NC, NS = 2, 16
NW = NC * NS
LB = 128
NTB = BATCH // LB
TB_PER_W = NTB // NW
NUNIT = TB_PER_W * HIST
DG = EMBED_DIM // 8


@functools.partial(
    pl.kernel,
    mesh=plsc.VectorSubcoreMesh(core_axis_name="c", subcore_axis_name="s"),
    out_type=jax.ShapeDtypeStruct((HIST, DG, NTB, 8, LB), jnp.float32),
    scratch_types=[
        pltpu.VMEM((TB_PER_W * LB, HIST), jnp.int32),
        pltpu.VMEM((3, LB), jnp.int32),
        pltpu.VMEM((LB, EMBED_DIM), jnp.float32),
        pltpu.VMEM((LB, EMBED_DIM), jnp.float32),
        pltpu.VMEM((LB, EMBED_DIM), jnp.float32),
        pltpu.VMEM((DG, 8, LB), jnp.float32),
        pltpu.VMEM((DG, 8, LB), jnp.float32),
        pltpu.VMEM((DG, 8, LB), jnp.float32),
        pltpu.SemaphoreType.DMA,
        pltpu.SemaphoreType.DMA,
        pltpu.SemaphoreType.DMA,
        pltpu.SemaphoreType.DMA,
        pltpu.SemaphoreType.DMA,
        pltpu.SemaphoreType.DMA,
    ],
    compiler_params=pltpu.CompilerParams(use_tc_tiling_on_sc=False,
                                         needs_layout_passes=False),
)
def _gather_kernel(x_hbm, table_hbm, out_hbm, xv, idxb,
                   rows0, rows1, rows2, st0, st1, st2,
                   g0, g1, g2, s0, s1, s2):
    rows = (rows0, rows1, rows2)
    stage = (st0, st1, st2)
    gsem = (g0, g1, g2)
    ssem = (s0, s1, s2)
    wid = lax.axis_index("s") * NC + lax.axis_index("c")

    pltpu.sync_copy(x_hbm.at[pl.ds(wid * (TB_PER_W * LB), TB_PER_W * LB)], xv)

    iot = lax.iota(jnp.int32, 16)

    def build_idx(b, u):
        tbl = u // HIST
        h = u % HIST
        hv = jnp.full((16,), h, jnp.int32)
        for lg in range(8):
            lv = iot + (tbl * LB + lg * 16)
            vals = plsc.load_gather(xv, [lv, hv])
            idxb[b, pl.ds(lg * 16, 16)] = vals

    def fire(b, u):
        build_idx(b, u)
        pltpu.async_copy(table_hbm.at[idxb.at[b]], rows[b], gsem[b])

    def drain_gather(b):
        pltpu.make_async_copy(
            table_hbm.at[idxb.at[b]], rows[b], gsem[b]
        ).wait()

    def transpose(b):
        rb = rows[b]
        sb = stage[b]

        def dbody(d, carry):
            g = d // 8
            e = d % 8
            dv = jnp.full((16,), d, jnp.int32)
            for lg in range(8):
                vals = plsc.load_gather(rb, [iot + lg * 16, dv])
                sb[g, e, pl.ds(lg * 16, 16)] = vals
            return carry

        lax.fori_loop(0, EMBED_DIM, dbody, 0)

    def fire_stores(b, u):
        tb = wid * TB_PER_W + u // HIST
        h = u % HIST
        for g in range(DG):
            pltpu.async_copy(stage[b].at[g], out_hbm.at[h, g, tb],
                             ssem[b])

    def wait_stores(b):
        for g in range(DG):
            pltpu.make_async_copy(
                stage[b].at[g], out_hbm.at[0, g, 0], ssem[b]
            ).wait()

    def visit(u, b, first3, last2):
        b2 = (b + 2) % 3
        if not last2:
            if first3:
                fire(b2, u + 2)
            else:
                @pl.when(u + 2 < NUNIT)
                def _():
                    fire(b2, u + 2)
        drain_gather(b)
        if not first3:
            wait_stores(b)
        transpose(b)
        fire_stores(b, u)

    fire(0, 0)
    fire(1, 1)
    visit(0, 0, True, False)
    visit(1, 1, True, False)
    visit(2, 2, True, False)

    def body(j, carry):
        u = 3 * j + 3
        for t in range(3):
            visit(u + t, t, False, False)
        return carry

    lax.fori_loop(0, 65, body, 0)

    visit(198, 0, False, True)
    visit(199, 1, False, True)
    wait_stores(2)
    wait_stores(0)
    wait_stores(1)


def kernel(x, table):
    out6 = _gather_kernel(x, table)
    return out6.transpose(2, 4, 0, 1, 3).reshape(BATCH, HIST, EMBED_DIM)


# speedup vs baseline: 1.1792x; 1.1792x over previous
import functools

import jax
import jax.numpy as jnp
from jax import lax
from jax.experimental import pallas as pl
from jax.experimental.pallas import tpu as pltpu
from jax.experimental.pallas import tpu_sc as plsc

VOCAB = 1000000
EMBED_DIM = 64
BATCH = 16384
HIST = 50

NC, NS = 2, 16
NW = NC * NS
LB = 128
NTB = BATCH // LB
TB_PER_W = NTB // NW
NUNIT = TB_PER_W * HIST
DG = EMBED_DIM // 8


@functools.partial(
    pl.kernel,
    mesh=plsc.VectorSubcoreMesh(core_axis_name="c", subcore_axis_name="s"),
    out_type=jax.ShapeDtypeStruct((HIST, DG, NTB, 8, LB), jnp.float32),
    scratch_types=[
        pltpu.VMEM((TB_PER_W * LB, HIST), jnp.int32),
        pltpu.VMEM((3, LB), jnp.int32),
        pltpu.VMEM((LB, EMBED_DIM), jnp.float32),
        pltpu.VMEM((LB, EMBED_DIM), jnp.float32),
        pltpu.VMEM((LB, EMBED_DIM), jnp.float32),
        pltpu.VMEM((DG, 8, LB), jnp.float32),
        pltpu.VMEM((DG, 8, LB), jnp.float32),
        pltpu.VMEM((DG, 8, LB), jnp.float32),
        pltpu.SemaphoreType.DMA,
        pltpu.SemaphoreType.DMA,
        pltpu.SemaphoreType.DMA,
        pltpu.SemaphoreType.DMA,
        pltpu.SemaphoreType.DMA,
        pltpu.SemaphoreType.DMA,
    ],
    compiler_params=pltpu.CompilerParams(use_tc_tiling_on_sc=False,
                                         needs_layout_passes=False),
)
def _gather_kernel(x_hbm, table_hbm, out_hbm, xv, idxb,
                   rows0, rows1, rows2, st0, st1, st2,
                   g0, g1, g2, s0, s1, s2):
    rows = (rows0, rows1, rows2)
    stage = (st0, st1, st2)
    gsem = (g0, g1, g2)
    ssem = (s0, s1, s2)
    wid = lax.axis_index("s") * NC + lax.axis_index("c")

    pltpu.sync_copy(x_hbm.at[pl.ds(wid * (TB_PER_W * LB), TB_PER_W * LB)], xv)

    iot = lax.iota(jnp.int32, 16)
    iots = tuple(iot + lg * 16 for lg in range(8))

    def build_idx(b, u):
        tbl = u // HIST
        h = u % HIST
        hv = jnp.full((16,), h, jnp.int32)
        vals = [plsc.load_gather(xv, [iots[lg] + tbl * LB, hv])
                for lg in range(8)]
        for lg in range(8):
            idxb[b, pl.ds(lg * 16, 16)] = vals[lg]

    def fire(b, u):
        build_idx(b, u)
        pltpu.async_copy(table_hbm.at[idxb.at[b]], rows[b], gsem[b])

    def drain_gather(b):
        pltpu.make_async_copy(
            table_hbm.at[idxb.at[b]], rows[b], gsem[b]
        ).wait()

    def transpose(b):
        rb = rows[b]
        sb = stage[b]

        def gbody(g, carry):
            for half in range(2):
                vals = []
                for e in range(half * 4, half * 4 + 4):
                    dv = jnp.full((16,), 8 * g + e, jnp.int32)
                    for lg in range(8):
                        vals.append(plsc.load_gather(rb, [iots[lg], dv]))
                k = 0
                for e in range(half * 4, half * 4 + 4):
                    for lg in range(8):
                        sb[g, e, pl.ds(lg * 16, 16)] = vals[k]
                        k += 1
            return carry

        lax.fori_loop(0, DG, gbody, 0)

    def fire_stores(b, u):
        tb = wid * TB_PER_W + u // HIST
        h = u % HIST
        for g in range(DG):
            pltpu.async_copy(stage[b].at[g], out_hbm.at[h, g, tb],
                             ssem[b])

    def wait_stores(b):
        for g in range(DG):
            pltpu.make_async_copy(
                stage[b].at[g], out_hbm.at[0, g, 0], ssem[b]
            ).wait()

    def visit(u, b, first3, last2):
        b2 = (b + 2) % 3
        if not last2:
            if first3:
                fire(b2, u + 2)
            else:
                @pl.when(u + 2 < NUNIT)
                def _():
                    fire(b2, u + 2)
        drain_gather(b)
        if not first3:
            wait_stores(b)
        transpose(b)
        fire_stores(b, u)

    fire(0, 0)
    fire(1, 1)
    visit(0, 0, True, False)
    visit(1, 1, True, False)
    visit(2, 2, True, False)

    def body(j, carry):
        u = 3 * j + 3
        for t in range(3):
            visit(u + t, t, False, False)
        return carry

    lax.fori_loop(0, 65, body, 0)

    visit(198, 0, False, True)
    visit(199, 1, False, True)
    wait_stores(2)
    wait_stores(0)
    wait_stores(1)


def kernel(x, table):
    out6 = _gather_kernel(x, table)
    return out6.transpose(2, 4, 0, 1, 3).reshape(BATCH, HIST, EMBED_DIM)


# speedup vs baseline: 1.4654x; 1.2427x over previous
import functools

import jax
import jax.numpy as jnp
from jax import lax
from jax.experimental import pallas as pl
from jax.experimental.pallas import tpu as pltpu
from jax.experimental.pallas import tpu_sc as plsc

VOCAB = 1000000
EMBED_DIM = 64
BATCH = 16384
HIST = 50

NC, NS = 2, 16
NW = NC * NS
B_PER_W = BATCH // NW
CB = 8
NCHUNK = B_PER_W // CB
NBUF = 3


@functools.partial(
    pl.kernel,
    mesh=plsc.VectorSubcoreMesh(core_axis_name="c", subcore_axis_name="s"),
    out_type=jax.ShapeDtypeStruct((BATCH, HIST, EMBED_DIM), jnp.float32),
    scratch_types=[
        pltpu.VMEM((B_PER_W, HIST), jnp.int32),
        pltpu.VMEM((CB, HIST, EMBED_DIM), jnp.float32),
        pltpu.VMEM((CB, HIST, EMBED_DIM), jnp.float32),
        pltpu.VMEM((CB, HIST, EMBED_DIM), jnp.float32),
        pltpu.SemaphoreType.DMA,
        pltpu.SemaphoreType.DMA,
        pltpu.SemaphoreType.DMA,
        pltpu.SemaphoreType.DMA,
        pltpu.SemaphoreType.DMA,
        pltpu.SemaphoreType.DMA,
    ],
    compiler_params=pltpu.CompilerParams(use_tc_tiling_on_sc=False),
)
def _gather_kernel(x_hbm, table_hbm, out_hbm, idx_v,
                   rows0, rows1, rows2, g0, g1, g2, s0, s1, s2):
    rows = (rows0, rows1, rows2)
    gsem = (g0, g1, g2)
    ssem = (s0, s1, s2)
    wid = lax.axis_index("s") * NC + lax.axis_index("c")
    bbase = wid * B_PER_W

    pltpu.sync_copy(x_hbm.at[pl.ds(bbase, B_PER_W)], idx_v)

    def fire(b, c):
        for t in range(CB):
            pltpu.async_copy(
                table_hbm.at[idx_v.at[c * CB + t]],
                rows[b].at[t],
                gsem[b],
            )

    def drain_store(b, c):
        pltpu.make_async_copy(
            out_hbm.at[pl.ds(bbase, CB)], rows[b], gsem[b]
        ).wait()
        pltpu.async_copy(rows[b], out_hbm.at[pl.ds(bbase + c * CB, CB)],
                         ssem[b])

    def wait_store(b):
        pltpu.make_async_copy(
            rows[b], out_hbm.at[pl.ds(bbase, CB)], ssem[b]
        ).wait()

    fire(0, 0)
    fire(1, 1)
    drain_store(0, 0)
    fire(2, 2)
    drain_store(1, 1)
    wait_store(0)
    fire(0, 3)

    def body(j, carry):
        c = 3 * j + 2
        for t in range(3):
            ct = c + t
            bt = (2 + t) % 3
            drain_store(bt, ct)
            wait_store((bt + 2) % 3)
            fire((bt + 2) % 3, ct + 2)
        return carry

    lax.fori_loop(0, 20, body, 0)

    drain_store(2, 62)
    wait_store(1)
    drain_store(0, 63)
    wait_store(2)
    wait_store(0)


def kernel(x, table):
    return _gather_kernel(x, table)
